# baseline (speedup 1.0000x reference)
import jax
import jax.numpy as jnp
from jax import lax
from jax.experimental import pallas as pl
from jax.experimental.pallas import tpu as pltpu
from jax.experimental.pallas import tpu_sc as plsc

N_FACTORS = 64
BATCH = 16384
CHUNK = 128
N_BLOCKS = 782
BLK_PER_W = 49
HALF = BATCH // 2
ELEM_PER_W = 512
LCAP = HALF + 80
DUMP = HALF


def _sweep(table_hbm, ids_all_v, list_v, bwork_v, blk_v, pack_v, win_e_v,
           staged, sid, sem):
    lanes = lax.iota(jnp.int32, 16)
    lo = sid * (BLK_PER_W * CHUNK)
    hi = lo + BLK_PER_W * CHUNK

    def build(g, ptr):
        v = ids_all_v[pl.ds(g * 16, 16)]
        m = (v >= lo) & (v < hi)
        packed = lax.shift_left(g * 16 + lanes, 17) | v
        plsc.store_compressed(list_v.at[pl.ds(ptr, 16)], packed, mask=m)
        pc = plsc.all_reduce_population_count(m)
        return ptr + pc[0]

    ptr = lax.fori_loop(0, HALF // 16, build, 0, unroll=8)
    n_groups = lax.shift_right_logical(ptr + 15, 4)

    def fire(j, p):
        t = jnp.minimum(sid * BLK_PER_W + j, N_BLOCKS - 1)
        off = pl.multiple_of(t * CHUNK, CHUNK)
        return pltpu.async_copy(
            table_hbm.at[:, pl.ds(off, CHUNK)],
            blk_v.at[pl.ds(p * 64, 64), :], sem)

    fire(0, 0)

    def block_body(j, carry):
        p = j & 1
        pltpu.make_async_copy(table_hbm.at[:, pl.ds(0, CHUNK)],
                              blk_v.at[pl.ds(0, 64), :], sem).wait()

        @pl.when(j < BLK_PER_W - 1)
        def _fire_next():
            fire(j + 1, 1 - p)

        t = sid * BLK_PER_W + j

        def refilter(g4, bptr):
            for q in range(4):
                g = g4 * 4 + q
                v = list_v[pl.ds(g * 16, 16)]
                m = (lax.shift_right_logical(v & 0x1FFFF, 7) == t)
                m = m & ((g * 16 + lanes) < ptr)
                plsc.store_compressed(bwork_v.at[pl.ds(bptr, 16)], v,
                                      mask=m)
                pc = plsc.all_reduce_population_count(m)
                bptr = bptr + pc[0]
            return bptr

        bptr = lax.fori_loop(0, lax.shift_right_logical(n_groups + 3, 2),
                             refilter, 0, unroll=False)
        n_win = lax.shift_right_logical(bptr + 15, 4)

        def window(w, _):
            v16 = bwork_v[pl.ds(w * 16, 16)]
            cols = v16 & 127
            es = lax.shift_right_logical(v16, 17)
            valid = (w * 16 + lanes) < bptr
            esel = jnp.where(valid, es, DUMP)
            win_e_v[0, :] = esel
            for k in range(16):
                ck = jnp.broadcast_to(cols[k], (16,))
                for c in range(4):
                    rows = p * 64 + c * 16 + lanes
                    pack_v[k, pl.ds(c * 16, 16)] = plsc.load_gather(
                        blk_v, [rows, ck])
            pltpu.sync_copy(pack_v, staged.at[win_e_v.at[0]])
            return _

        lax.fori_loop(0, n_win, window, 0, unroll=False)
        return carry

    lax.fori_loop(0, BLK_PER_W, block_body, 0, unroll=False)


def _mf_kernel(uid_hbm, iid_hbm, uembt_hbm, iembt_hbm, ubias_hbm, ibias_hbm,
               gbias_hbm, out_hbm, park_hbm,
               ids_all_v, list_v, bwork_v, blk_v, pack_v, win_e_v,
               uidc_v, iidc_v, blk2_v, ub_v, ib_v, gb_v, out_v,
               staged, sem, bsem):
    core = lax.axis_index("c")
    sid = lax.axis_index("s")
    ebase = core * HALF + sid * ELEM_PER_W

    n_chunks = ELEM_PER_W // CHUNK
    for j in range(n_chunks):
        pltpu.sync_copy(uid_hbm.at[pl.ds(ebase + j * CHUNK, CHUNK)],
                        uidc_v.at[j])
        pltpu.sync_copy(iid_hbm.at[pl.ds(ebase + j * CHUNK, CHUNK)],
                        iidc_v.at[j])
    pltpu.sync_copy(gbias_hbm, gb_v)

    bias_copies = []
    for j in range(n_chunks):
        sl = pl.ds(j * CHUNK, CHUNK)
        bias_copies.append(pltpu.async_copy(ubias_hbm.at[uidc_v.at[j]],
                                            ub_v.at[sl], bsem))
        bias_copies.append(pltpu.async_copy(ibias_hbm.at[iidc_v.at[j]],
                                            ib_v.at[sl], bsem))

    pltpu.sync_copy(uid_hbm.at[pl.ds(core * HALF, HALF)], ids_all_v)
    _sweep(uembt_hbm, ids_all_v, list_v, bwork_v, blk_v, pack_v, win_e_v,
           staged, sid, sem)
    plsc.subcore_barrier()
    pltpu.sync_copy(staged.at[pl.ds(sid * ELEM_PER_W, ELEM_PER_W)],
                    park_hbm.at[pl.ds(ebase, ELEM_PER_W)])
    plsc.subcore_barrier()
    pltpu.sync_copy(iid_hbm.at[pl.ds(core * HALF, HALF)], ids_all_v)
    _sweep(iembt_hbm, ids_all_v, list_v, bwork_v, blk_v, pack_v, win_e_v,
           staged, sid, sem)
    plsc.subcore_barrier()

    for c in bias_copies:
        c.wait()

    gvec = gb_v[...]
    lanes = lax.iota(jnp.int32, 16)

    for q in range(ELEM_PER_W // CHUNK):
        pltpu.sync_copy(
            staged.at[pl.ds(sid * ELEM_PER_W + q * CHUNK, CHUNK)],
            blk_v.at[pl.ds(0, CHUNK), :])
        pltpu.sync_copy(park_hbm.at[pl.ds(ebase + q * CHUNK, CHUNK)],
                        blk2_v)

        def group_body(g, _, q=q):
            loff = g * 16
            off = q * CHUNK + loff
            lrows = loff + lanes
            acc = ub_v[pl.ds(off, 16)] + ib_v[pl.ds(off, 16)] + gvec
            for d in range(N_FACTORS):
                col = jnp.full((16,), d, jnp.int32)
                u = plsc.load_gather(blk2_v, [lrows, col])
                v = plsc.load_gather(blk_v, [lrows, col])
                acc = acc + u * v
            out_v[pl.ds(off, 16)] = acc
            return _

        lax.fori_loop(0, CHUNK // 16, group_body, 0, unroll=False)

    pltpu.sync_copy(out_v, out_hbm.at[pl.ds(ebase, ELEM_PER_W)])


def kernel(user_ids, item_ids, user_embedding, item_embedding, user_bias,
           item_bias, global_bias):
    uid = user_ids.astype(jnp.int32)
    iid = item_ids.astype(jnp.int32)
    uet = user_embedding.T
    iet = item_embedding.T
    ub = user_bias.reshape(-1)
    ib = item_bias.reshape(-1)
    gb = jnp.broadcast_to(global_bias.astype(jnp.float32), (16,))

    n_chunks = ELEM_PER_W // CHUNK
    mesh = plsc.VectorSubcoreMesh(core_axis_name="c", subcore_axis_name="s")
    f = pl.kernel(
        _mf_kernel,
        mesh=mesh,
        compiler_params=pltpu.CompilerParams(needs_layout_passes=False,
                                             use_tc_tiling_on_sc=True),
        out_type=(jax.ShapeDtypeStruct((BATCH,), jnp.float32),
                  jax.ShapeDtypeStruct((BATCH, CHUNK), jnp.float32)),
        scratch_types=[
            pltpu.VMEM((HALF,), jnp.int32),
            pltpu.VMEM((LCAP,), jnp.int32),
            pltpu.VMEM((LCAP,), jnp.int32),
            pltpu.VMEM((128, CHUNK), jnp.float32),
            pltpu.VMEM((16, CHUNK), jnp.float32),
            pltpu.VMEM((1, 16), jnp.int32),
            pltpu.VMEM((n_chunks, CHUNK), jnp.int32),
            pltpu.VMEM((n_chunks, CHUNK), jnp.int32),
            pltpu.VMEM((CHUNK, CHUNK), jnp.float32),
            pltpu.VMEM((ELEM_PER_W,), jnp.float32),
            pltpu.VMEM((ELEM_PER_W,), jnp.float32),
            pltpu.VMEM((16,), jnp.float32),
            pltpu.VMEM((ELEM_PER_W,), jnp.float32),
            pltpu.VMEM_SHARED((HALF + 8, CHUNK), jnp.float32),
            pltpu.SemaphoreType.DMA,
            pltpu.SemaphoreType.DMA,
        ],
    )
    return f(uid, iid, uet, iet, ub, ib, gb)[0]


# speedup vs baseline: 1.3146x; 1.3146x over previous
import jax
import jax.numpy as jnp
from jax import lax
from jax.experimental import pallas as pl
from jax.experimental.pallas import tpu as pltpu
from jax.experimental.pallas import tpu_sc as plsc

N_FACTORS = 64
BATCH = 16384
CHUNK = 128


def _mf_kernel(uid_hbm, iid_hbm, uemb_hbm, iemb_hbm, ubias_hbm, ibias_hbm,
               gbias_hbm, out_hbm,
               uidx_v, iidx_v, urows_v, irows_v, ub_v, ib_v, gb_v, out_v,
               sem):
    info = plsc.get_sparse_core_info()
    nc = info.num_cores
    wid = lax.axis_index("s") * nc + lax.axis_index("c")
    n_chunks = uidx_v.shape[0]
    b_per_w = n_chunks * CHUNK
    base = wid * b_per_w

    for j in range(n_chunks):
        pltpu.sync_copy(uid_hbm.at[pl.ds(base + j * CHUNK, CHUNK)],
                        uidx_v.at[j])
        pltpu.sync_copy(iid_hbm.at[pl.ds(base + j * CHUNK, CHUNK)],
                        iidx_v.at[j])
    pltpu.sync_copy(gbias_hbm, gb_v)

    copies = []
    for j in range(n_chunks):
        sl = pl.ds(j * CHUNK, CHUNK)
        copies.append(pltpu.async_copy(uemb_hbm.at[uidx_v.at[j]],
                                       urows_v.at[sl], sem))
        copies.append(pltpu.async_copy(iemb_hbm.at[iidx_v.at[j]],
                                       irows_v.at[sl], sem))
        copies.append(pltpu.async_copy(ubias_hbm.at[uidx_v.at[j]],
                                       ub_v.at[sl], sem))
        copies.append(pltpu.async_copy(ibias_hbm.at[iidx_v.at[j]],
                                       ib_v.at[sl], sem))
    for c in copies:
        c.wait()

    gvec = gb_v[...]
    lanes = lax.iota(jnp.int32, 16)

    def group_body(g, _):
        off = g * 16
        rows = off + lanes
        acc = ub_v[pl.ds(off, 16)] + ib_v[pl.ds(off, 16)] + gvec
        for d in range(N_FACTORS):
            col = jnp.full((16,), d, jnp.int32)
            u = plsc.load_gather(urows_v, [rows, col])
            v = plsc.load_gather(irows_v, [rows, col])
            acc = acc + u * v
        out_v[pl.ds(off, 16)] = acc
        return _

    lax.fori_loop(0, b_per_w // 16, group_body, 0, unroll=False)

    pltpu.sync_copy(out_v, out_hbm.at[pl.ds(base, b_per_w)])


def kernel(user_ids, item_ids, user_embedding, item_embedding, user_bias,
           item_bias, global_bias):
    nw = 32
    b_per_w = BATCH // nw
    n_chunks = b_per_w // CHUNK

    uid = user_ids.astype(jnp.int32)
    iid = item_ids.astype(jnp.int32)
    ub = user_bias.reshape(-1)
    ib = item_bias.reshape(-1)
    gb = jnp.broadcast_to(global_bias.astype(jnp.float32), (16,))

    mesh = plsc.VectorSubcoreMesh(core_axis_name="c", subcore_axis_name="s")
    f = pl.kernel(
        _mf_kernel,
        mesh=mesh,
        compiler_params=pltpu.CompilerParams(needs_layout_passes=False,
                                             use_tc_tiling_on_sc=False),
        out_type=jax.ShapeDtypeStruct((BATCH,), jnp.float32),
        scratch_types=[
            pltpu.VMEM((n_chunks, CHUNK), jnp.int32),
            pltpu.VMEM((n_chunks, CHUNK), jnp.int32),
            pltpu.VMEM((b_per_w, N_FACTORS), jnp.float32),
            pltpu.VMEM((b_per_w, N_FACTORS), jnp.float32),
            pltpu.VMEM((b_per_w,), jnp.float32),
            pltpu.VMEM((b_per_w,), jnp.float32),
            pltpu.VMEM((16,), jnp.float32),
            pltpu.VMEM((b_per_w,), jnp.float32),
            pltpu.SemaphoreType.DMA,
        ],
    )
    return f(uid, iid, user_embedding, item_embedding, ub, ib, gb)


# speedup vs baseline: 1.3529x; 1.0291x over previous
import jax
import jax.numpy as jnp
from jax import lax
from jax.experimental import pallas as pl
from jax.experimental.pallas import tpu as pltpu
from jax.experimental.pallas import tpu_sc as plsc

N_FACTORS = 64
BATCH = 16384
CHUNK = 128


def _mf_kernel(uid_hbm, iid_hbm, uemb_hbm, iemb_hbm, ubias_hbm, ibias_hbm,
               gbias_hbm, out_hbm,
               uidx_v, iidx_v, urows_v, irows_v, ub_v, ib_v, gb_v, out_v,
               sem):
    info = plsc.get_sparse_core_info()
    nc = info.num_cores
    wid = lax.axis_index("s") * nc + lax.axis_index("c")
    n_chunks = uidx_v.shape[0]
    b_per_w = n_chunks * CHUNK
    base = wid * b_per_w

    for j in range(n_chunks):
        pltpu.sync_copy(uid_hbm.at[pl.ds(base + j * CHUNK, CHUNK)],
                        uidx_v.at[j])
        pltpu.sync_copy(iid_hbm.at[pl.ds(base + j * CHUNK, CHUNK)],
                        iidx_v.at[j])
    pltpu.sync_copy(gbias_hbm, gb_v)

    bias_copies = []
    for j in range(n_chunks):
        sl = pl.ds(j * CHUNK, CHUNK)
        bias_copies.append(pltpu.async_copy(ubias_hbm.at[uidx_v.at[j]],
                                            ub_v.at[sl], sem))
        bias_copies.append(pltpu.async_copy(ibias_hbm.at[iidx_v.at[j]],
                                            ib_v.at[sl], sem))

    gvec = gb_v[...]
    lanes = lax.iota(jnp.int32, 16)
    half_chunks = n_chunks // 2

    for h in range(2):
        copies = []
        for jj in range(half_chunks):
            j = h * half_chunks + jj
            sl = pl.ds(jj * CHUNK, CHUNK)
            copies.append(pltpu.async_copy(uemb_hbm.at[uidx_v.at[j]],
                                           urows_v.at[sl], sem))
            copies.append(pltpu.async_copy(iemb_hbm.at[iidx_v.at[j]],
                                           irows_v.at[sl], sem))
        for c in copies:
            c.wait()
        if h == 0:
            for c in bias_copies:
                c.wait()

        def group_body(g, _, h=h):
            loff = g * 16
            off = h * (half_chunks * CHUNK) + loff
            rows = loff + lanes
            acc = ub_v[pl.ds(off, 16)] + ib_v[pl.ds(off, 16)] + gvec
            for d in range(N_FACTORS):
                col = jnp.full((16,), d, jnp.int32)
                u = plsc.load_gather(urows_v, [rows, col])
                v = plsc.load_gather(irows_v, [rows, col])
                acc = acc + u * v
            out_v[pl.ds(off, 16)] = acc
            return _

        lax.fori_loop(0, half_chunks * CHUNK // 16, group_body, 0,
                      unroll=False)

    pltpu.sync_copy(out_v, out_hbm.at[pl.ds(base, b_per_w)])


def kernel(user_ids, item_ids, user_embedding, item_embedding, user_bias,
           item_bias, global_bias):
    nw = 32
    b_per_w = BATCH // nw
    n_chunks = b_per_w // CHUNK

    uid = user_ids.astype(jnp.int32)
    iid = item_ids.astype(jnp.int32)
    ue2 = jnp.pad(user_embedding, ((0, 0), (0, CHUNK - N_FACTORS)))
    ie2 = jnp.pad(item_embedding, ((0, 0), (0, CHUNK - N_FACTORS)))
    ub = user_bias.reshape(-1)
    ib = item_bias.reshape(-1)
    gb = jnp.broadcast_to(global_bias.astype(jnp.float32), (16,))

    mesh = plsc.VectorSubcoreMesh(core_axis_name="c", subcore_axis_name="s")
    f = pl.kernel(
        _mf_kernel,
        mesh=mesh,
        compiler_params=pltpu.CompilerParams(needs_layout_passes=False,
                                             use_tc_tiling_on_sc=True),
        out_type=jax.ShapeDtypeStruct((BATCH,), jnp.float32),
        scratch_types=[
            pltpu.VMEM((n_chunks, CHUNK), jnp.int32),
            pltpu.VMEM((n_chunks, CHUNK), jnp.int32),
            pltpu.VMEM((b_per_w // 2, CHUNK), jnp.float32),
            pltpu.VMEM((b_per_w // 2, CHUNK), jnp.float32),
            pltpu.VMEM((b_per_w,), jnp.float32),
            pltpu.VMEM((b_per_w,), jnp.float32),
            pltpu.VMEM((16,), jnp.float32),
            pltpu.VMEM((b_per_w,), jnp.float32),
            pltpu.SemaphoreType.DMA,
        ],
    )
    return f(uid, iid, ue2, ie2, ub, ib, gb)


# speedup vs baseline: 1.3841x; 1.0230x over previous
import jax
import jax.numpy as jnp
from jax import lax
from jax.experimental import pallas as pl
from jax.experimental.pallas import tpu as pltpu
from jax.experimental.pallas import tpu_sc as plsc

N_FACTORS = 64
BATCH = 16384
CHUNK = 128


def _mf_kernel(uid_hbm, iid_hbm, uemb_hbm, iemb_hbm, ubias_hbm, ibias_hbm,
               gbias_hbm, out_hbm,
               uidx_v, iidx_v, urows_v, irows_v, ub_v, ib_v, gb_v, out_v,
               sem):
    info = plsc.get_sparse_core_info()
    nc = info.num_cores
    wid = lax.axis_index("s") * nc + lax.axis_index("c")
    n_chunks = uidx_v.shape[0]
    b_per_w = n_chunks * CHUNK
    base = wid * b_per_w

    for j in range(n_chunks):
        pltpu.sync_copy(uid_hbm.at[pl.ds(base + j * CHUNK, CHUNK)],
                        uidx_v.at[j])
        pltpu.sync_copy(iid_hbm.at[pl.ds(base + j * CHUNK, CHUNK)],
                        iidx_v.at[j])
    pltpu.sync_copy(gbias_hbm, gb_v)

    bias_copies = []
    for j in range(n_chunks):
        sl = pl.ds(j * CHUNK, CHUNK)
        bias_copies.append(pltpu.async_copy(ubias_hbm.at[uidx_v.at[j]],
                                            ub_v.at[sl], sem))
        bias_copies.append(pltpu.async_copy(ibias_hbm.at[iidx_v.at[j]],
                                            ib_v.at[sl], sem))

    gvec = gb_v[...]
    lanes = lax.iota(jnp.int32, 16)

    u_copies = []
    for j in range(n_chunks):
        sl = pl.ds(j * CHUNK, CHUNK)
        u_copies.append(pltpu.async_copy(uemb_hbm.at[uidx_v.at[j]],
                                         urows_v.at[sl], sem))

    def fire_item(j):
        return pltpu.async_copy(iemb_hbm.at[iidx_v.at[j]],
                                irows_v.at[pl.ds((j % 2) * CHUNK, CHUNK)],
                                sem)

    i_copies = {0: fire_item(0), 1: fire_item(1)}

    for j in range(n_chunks):
        u_copies[j].wait()
        i_copies[j].wait()
        if j == 0:
            for c in bias_copies:
                c.wait()

        def group_body(g, _, j=j):
            off = j * CHUNK + g * 16
            rows = off + lanes
            lrows = (j % 2) * CHUNK + g * 16 + lanes
            acc = ub_v[pl.ds(off, 16)] + ib_v[pl.ds(off, 16)] + gvec
            for d in range(N_FACTORS):
                col = jnp.full((16,), d, jnp.int32)
                u = plsc.load_gather(urows_v, [rows, col])
                v = plsc.load_gather(irows_v, [lrows, col])
                acc = acc + u * v
            out_v[pl.ds(off, 16)] = acc
            return _

        lax.fori_loop(0, CHUNK // 16, group_body, 0, unroll=False)
        if j + 2 < n_chunks:
            i_copies[j + 2] = fire_item(j + 2)

    pltpu.sync_copy(out_v, out_hbm.at[pl.ds(base, b_per_w)])


def kernel(user_ids, item_ids, user_embedding, item_embedding, user_bias,
           item_bias, global_bias):
    nw = 32
    b_per_w = BATCH // nw
    n_chunks = b_per_w // CHUNK

    uid = user_ids.astype(jnp.int32)
    iid = item_ids.astype(jnp.int32)
    ue2 = jnp.pad(user_embedding, ((0, 0), (0, CHUNK - N_FACTORS)))
    ie2 = jnp.pad(item_embedding, ((0, 0), (0, CHUNK - N_FACTORS)))
    ub = user_bias.reshape(-1)
    ib = item_bias.reshape(-1)
    gb = jnp.broadcast_to(global_bias.astype(jnp.float32), (16,))

    mesh = plsc.VectorSubcoreMesh(core_axis_name="c", subcore_axis_name="s")
    f = pl.kernel(
        _mf_kernel,
        mesh=mesh,
        compiler_params=pltpu.CompilerParams(needs_layout_passes=False,
                                             use_tc_tiling_on_sc=True),
        out_type=jax.ShapeDtypeStruct((BATCH,), jnp.float32),
        scratch_types=[
            pltpu.VMEM((n_chunks, CHUNK), jnp.int32),
            pltpu.VMEM((n_chunks, CHUNK), jnp.int32),
            pltpu.VMEM((b_per_w, CHUNK), jnp.float32),
            pltpu.VMEM((2 * CHUNK, CHUNK), jnp.float32),
            pltpu.VMEM((b_per_w,), jnp.float32),
            pltpu.VMEM((b_per_w,), jnp.float32),
            pltpu.VMEM((16,), jnp.float32),
            pltpu.VMEM((b_per_w,), jnp.float32),
            pltpu.SemaphoreType.DMA,
        ],
    )
    return f(uid, iid, ue2, ie2, ub, ib, gb)
